# baseline (speedup 1.0000x reference)
import functools

import jax
import jax.numpy as jnp
from jax import lax
from jax.experimental import pallas as pl
from jax.experimental.pallas import tpu as pltpu
from jax.experimental.pallas import tpu_sc as plsc

B = 1024
F = 26
K = 16
V = 1000000

NC = 2
NS = 16
NW = NC * NS
SAMP_PER_W = B // NW
IDX_PER_W = SAMP_PER_W * F
CHUNK = 104
NCHUNK = IDX_PER_W // CHUNK


TPW = 16
NWIN = 16
PSTRIDE = 7840


def _sc_detile_body(emb_hbm, out_hbm, buf0, buf1, sem, sem_w):
    wid = lax.axis_index("s") * NC + lax.axis_index("c")
    base = jnp.minimum(NWIN * wid, 488 - NWIN)
    bufs = (buf0, buf1)

    def window(i, carry):
        t0 = (base + i) * TPW
        wrs = []
        for k1 in range(2):
            buf = bufs[k1]
            rds = []
            for tl in range(TPW):
                col = pl.multiple_of((t0 + tl) * 128, 128)
                rds.append(pltpu.async_copy(
                    emb_hbm.at[pl.ds(k1 * 8, 8), pl.ds(col, 128)],
                    buf.at[tl], sem))
            for h in rds:
                h.wait()
            for k2 in range(8):
                row0 = (k1 * 8 + k2) * PSTRIDE + t0
                wrs.append(pltpu.async_copy(
                    buf.at[:, k2], out_hbm.at[pl.ds(row0, TPW)], sem_w))
        for h in wrs:
            h.wait()
        return carry

    lax.fori_loop(0, NWIN, window, 0)

    for k1 in range(2):
        buf = bufs[k1]
        rds = []
        for tl in range(4):
            col = pl.multiple_of((7808 + tl) * 128, 128)
            rds.append(pltpu.async_copy(
                emb_hbm.at[pl.ds(k1 * 8, 8), pl.ds(col, 128)],
                buf.at[tl], sem))
        for h in rds:
            h.wait()
        wrs = []
        for k2 in range(8):
            row0 = (k1 * 8 + k2) * PSTRIDE + 7808
            wrs.append(pltpu.async_copy(
                buf.at[:, k2], out_hbm.at[pl.ds(row0, TPW)], sem_w))
        for h in wrs:
            h.wait()


@jax.jit
def _sc_detile(emb_t):
    run = functools.partial(
        pl.kernel,
        mesh=plsc.VectorSubcoreMesh(core_axis_name="c", subcore_axis_name="s"),
        out_type=jax.ShapeDtypeStruct((K * PSTRIDE, 128), jnp.float32),
        scratch_types=[
            pltpu.VMEM((TPW, 8, 128), jnp.float32),
            pltpu.VMEM((TPW, 8, 128), jnp.float32),
            pltpu.SemaphoreType.DMA,
            pltpu.SemaphoreType.DMA,
        ],
        compiler_params=pltpu.CompilerParams(use_tc_tiling_on_sc=True),
    )(_sc_detile_body)
    return run(emb_t)


def _sc_body(x_hbm, emb_hbm, bias_hbm, w0_hbm, ea_out, ep_out,
             idx_v, val_v, bv_v, ea_v, ep_v, w0_v, sem):
    wid = lax.axis_index("s") * NC + lax.axis_index("c")
    sw = wid * SAMP_PER_W

    pltpu.sync_copy(w0_hbm, w0_v)
    idx_cps = [
        pltpu.async_copy(x_hbm.at[f, pl.ds(sw, SAMP_PER_W)],
                         idx_v.at[pl.ds(f * SAMP_PER_W, SAMP_PER_W)], sem)
        for f in range(F)
    ]
    for cp in idx_cps:
        cp.wait()

    copies = []
    for c in range(NCHUNK):
        sl = pl.ds(c * CHUNK, CHUNK)
        for k in range(K):
            copies.append(pltpu.async_copy(
                emb_hbm.at[k].at[idx_v.at[sl]],
                val_v.at[k, sl], sem))
        copies.append(pltpu.async_copy(
            bias_hbm.at[idx_v.at[sl]], bv_v.at[sl], sem))
    for cp in copies:
        cp.wait()

    w0v = w0_v[...]

    for h in range(SAMP_PER_W // 16):
        def eabody(f, carry):
            sl = pl.ds(f * SAMP_PER_W + h * 16, 16)
            ea_v[f, pl.ds(h * 16, 16)] = jnp.exp(-(w0v + bv_v[sl]))
            return carry
        lax.fori_loop(0, F, eabody, 0)

    for sb in range(SAMP_PER_W // 16):
        def fbody(f, accs):
            new = []
            for k in range(K):
                val = val_v[k, pl.ds(f * SAMP_PER_W + sb * 16, 16)]
                acc, asq = accs[2 * k], accs[2 * k + 1]
                new.append(acc + val)
                new.append(asq + val * val)
            return tuple(new)

        zero = jnp.zeros((16,), jnp.float32)
        accs = lax.fori_loop(0, F, fbody, (zero,) * (2 * K))
        u = zero
        for k in range(K):
            acc, asq = accs[2 * k], accs[2 * k + 1]
            u = u + (acc * acc - asq)
        ep_v[pl.ds(sb * 16, 16)] = jnp.exp(-0.5 * u)

    pltpu.sync_copy(ea_v, ea_out.at[:, pl.ds(sw, SAMP_PER_W)])
    pltpu.sync_copy(ep_v, ep_out.at[pl.ds(sw, SAMP_PER_W)])


@jax.jit
def _sc_gather_reduce(x2d, emb3, bias_lin, w016):
    run = functools.partial(
        pl.kernel,
        mesh=plsc.VectorSubcoreMesh(core_axis_name="c", subcore_axis_name="s"),
        out_type=[
            jax.ShapeDtypeStruct((F, B), jnp.float32),
            jax.ShapeDtypeStruct((B,), jnp.float32),
        ],
        scratch_types=[
            pltpu.VMEM((IDX_PER_W,), jnp.int32),
            pltpu.VMEM((K, IDX_PER_W), jnp.float32),
            pltpu.VMEM((IDX_PER_W,), jnp.float32),
            pltpu.VMEM((F, SAMP_PER_W), jnp.float32),
            pltpu.VMEM((SAMP_PER_W,), jnp.float32),
            pltpu.VMEM((16,), jnp.float32),
            pltpu.SemaphoreType.DMA,
        ],
        compiler_params=pltpu.CompilerParams(use_tc_tiling_on_sc=False),
    )(_sc_body)
    return run(x2d, emb3, bias_lin, w016)


BB = 128


def _tc_body(ea_ref, ep_ref, out_ref):
    ea = ea_ref[...]
    ep = ep_ref[...][0]
    x = ea[:, :, None] * ep[None, None, :]
    out_ref[...] = 5.5 / (1.0 + x)


@jax.jit
def _tc_broadcast(ea2d, ep2d):
    return pl.pallas_call(
        _tc_body,
        grid=(B // BB,),
        in_specs=[
            pl.BlockSpec((F, BB), lambda i: (0, i)),
            pl.BlockSpec((1, B), lambda i: (0, 0)),
        ],
        out_specs=pl.BlockSpec((F, BB, B), lambda i: (0, i, 0)),
        out_shape=jax.ShapeDtypeStruct((F, B, B), jnp.float32),
    )(ea2d, ep2d)


def kernel(X, emb_table, bias_table, w0):
    x2d = X.T.astype(jnp.int32)
    det3 = _sc_detile(emb_table.T).reshape(K, PSTRIDE, 128)
    tail = emb_table[999936:].T[:, None, :]
    det3 = lax.dynamic_update_slice(det3, tail, (0, 7812, 0))
    emb_pad = det3.reshape(K, PSTRIDE * 128)
    bias_lin = bias_table.reshape(V)
    w016 = jnp.broadcast_to(w0.astype(jnp.float32), (16,))
    ea2d, ep = _sc_gather_reduce(x2d, emb_pad, bias_lin, w016)
    out3 = _tc_broadcast(ea2d, ep.reshape(1, B))
    return out3.transpose(1, 0, 2)


# speedup vs baseline: 1.0766x; 1.0766x over previous
import functools

import jax
import jax.numpy as jnp
from jax import lax
from jax.experimental import pallas as pl
from jax.experimental.pallas import tpu as pltpu
from jax.experimental.pallas import tpu_sc as plsc

B = 1024
F = 26
K = 16
V = 1000000

NC = 2
NS = 16
NW = NC * NS
SAMP_PER_W = B // NW
IDX_PER_W = SAMP_PER_W * F
CHUNK = 104
NCHUNK = IDX_PER_W // CHUNK


TPW = 16
NWIN = 16
PSTRIDE = 7840


def _sc_detile_body(emb_hbm, out_hbm, buf0, buf1, buf2, buf3, sem, sem_w):
    wid = lax.axis_index("s") * NC + lax.axis_index("c")
    base = jnp.minimum(NWIN * wid, 488 - NWIN)
    rings = ((buf0, buf1), (buf2, buf3))

    def _drain_one_window(n):
        for _ in range(16 * n):
            pltpu.make_async_copy(out_hbm.at[pl.ds(0, TPW)],
                                  buf0.at[:, 0], sem_w).wait()

    def pair(g, carry):
        for p in range(2):
            t0 = (base + 2 * g + p) * TPW

            @pl.when(g > 0)
            def _():
                _drain_one_window(1)

            rds = []
            for k1 in range(2):
                buf = rings[p][k1]
                for tl in range(TPW):
                    col = pl.multiple_of((t0 + tl) * 128, 128)
                    rds.append(pltpu.async_copy(
                        emb_hbm.at[pl.ds(k1 * 8, 8), pl.ds(col, 128)],
                        buf.at[tl], sem))
            for h in rds:
                h.wait()
            for k1 in range(2):
                buf = rings[p][k1]
                for k2 in range(8):
                    row0 = (k1 * 8 + k2) * PSTRIDE + t0
                    pltpu.async_copy(
                        buf.at[:, k2], out_hbm.at[pl.ds(row0, TPW)], sem_w)
        return carry

    lax.fori_loop(0, NWIN // 2, pair, 0)
    _drain_one_window(2)

    for k1 in range(2):
        buf = rings[0][k1]
        rds = []
        for tl in range(4):
            col = pl.multiple_of((7808 + tl) * 128, 128)
            rds.append(pltpu.async_copy(
                emb_hbm.at[pl.ds(k1 * 8, 8), pl.ds(col, 128)],
                buf.at[tl], sem))
        for h in rds:
            h.wait()
        wrs = []
        for k2 in range(8):
            row0 = (k1 * 8 + k2) * PSTRIDE + 7808
            wrs.append(pltpu.async_copy(
                buf.at[:, k2], out_hbm.at[pl.ds(row0, TPW)], sem_w))
        for h in wrs:
            h.wait()


@jax.jit
def _sc_detile(emb_t):
    run = functools.partial(
        pl.kernel,
        mesh=plsc.VectorSubcoreMesh(core_axis_name="c", subcore_axis_name="s"),
        out_type=jax.ShapeDtypeStruct((K * PSTRIDE, 128), jnp.float32),
        scratch_types=[
            pltpu.VMEM((TPW, 8, 128), jnp.float32),
            pltpu.VMEM((TPW, 8, 128), jnp.float32),
            pltpu.VMEM((TPW, 8, 128), jnp.float32),
            pltpu.VMEM((TPW, 8, 128), jnp.float32),
            pltpu.SemaphoreType.DMA,
            pltpu.SemaphoreType.DMA,
        ],
        compiler_params=pltpu.CompilerParams(use_tc_tiling_on_sc=True),
    )(_sc_detile_body)
    return run(emb_t)


def _sc_body(x_hbm, emb_hbm, bias_hbm, w0_hbm, ea_out, ep_out,
             idx_v, val_v, bv_v, ea_v, ep_v, w0_v, sem):
    wid = lax.axis_index("s") * NC + lax.axis_index("c")
    sw = wid * SAMP_PER_W

    pltpu.sync_copy(w0_hbm, w0_v)
    idx_cps = [
        pltpu.async_copy(x_hbm.at[f, pl.ds(sw, SAMP_PER_W)],
                         idx_v.at[pl.ds(f * SAMP_PER_W, SAMP_PER_W)], sem)
        for f in range(F)
    ]
    for cp in idx_cps:
        cp.wait()

    copies = []
    for c in range(NCHUNK):
        sl = pl.ds(c * CHUNK, CHUNK)
        for k in range(K):
            copies.append(pltpu.async_copy(
                emb_hbm.at[k].at[idx_v.at[sl]],
                val_v.at[k, sl], sem))
        copies.append(pltpu.async_copy(
            bias_hbm.at[idx_v.at[sl]], bv_v.at[sl], sem))
    for cp in copies:
        cp.wait()

    w0v = w0_v[...]

    for h in range(SAMP_PER_W // 16):
        def eabody(f, carry):
            sl = pl.ds(f * SAMP_PER_W + h * 16, 16)
            ea_v[f, pl.ds(h * 16, 16)] = jnp.exp(-(w0v + bv_v[sl]))
            return carry
        lax.fori_loop(0, F, eabody, 0)

    for sb in range(SAMP_PER_W // 16):
        def fbody(f, accs):
            new = []
            for k in range(K):
                val = val_v[k, pl.ds(f * SAMP_PER_W + sb * 16, 16)]
                acc, asq = accs[2 * k], accs[2 * k + 1]
                new.append(acc + val)
                new.append(asq + val * val)
            return tuple(new)

        zero = jnp.zeros((16,), jnp.float32)
        accs = lax.fori_loop(0, F, fbody, (zero,) * (2 * K))
        u = zero
        for k in range(K):
            acc, asq = accs[2 * k], accs[2 * k + 1]
            u = u + (acc * acc - asq)
        ep_v[pl.ds(sb * 16, 16)] = jnp.exp(-0.5 * u)

    pltpu.sync_copy(ea_v, ea_out.at[:, pl.ds(sw, SAMP_PER_W)])
    pltpu.sync_copy(ep_v, ep_out.at[pl.ds(sw, SAMP_PER_W)])


@jax.jit
def _sc_gather_reduce(x2d, emb3, bias_lin, w016):
    run = functools.partial(
        pl.kernel,
        mesh=plsc.VectorSubcoreMesh(core_axis_name="c", subcore_axis_name="s"),
        out_type=[
            jax.ShapeDtypeStruct((F, B), jnp.float32),
            jax.ShapeDtypeStruct((B,), jnp.float32),
        ],
        scratch_types=[
            pltpu.VMEM((IDX_PER_W,), jnp.int32),
            pltpu.VMEM((K, IDX_PER_W), jnp.float32),
            pltpu.VMEM((IDX_PER_W,), jnp.float32),
            pltpu.VMEM((F, SAMP_PER_W), jnp.float32),
            pltpu.VMEM((SAMP_PER_W,), jnp.float32),
            pltpu.VMEM((16,), jnp.float32),
            pltpu.SemaphoreType.DMA,
        ],
        compiler_params=pltpu.CompilerParams(use_tc_tiling_on_sc=False),
    )(_sc_body)
    return run(x2d, emb3, bias_lin, w016)


BB = 128


def _tc_body(ea_ref, ep_ref, out_ref):
    ea = ea_ref[...]
    ep = ep_ref[...][0]
    x = ea[:, :, None] * ep[None, None, :]
    out_ref[...] = 5.5 / (1.0 + x)


@jax.jit
def _tc_broadcast(ea2d, ep2d):
    return pl.pallas_call(
        _tc_body,
        grid=(B // BB,),
        in_specs=[
            pl.BlockSpec((F, BB), lambda i: (0, i)),
            pl.BlockSpec((1, B), lambda i: (0, 0)),
        ],
        out_specs=pl.BlockSpec((F, BB, B), lambda i: (0, i, 0)),
        out_shape=jax.ShapeDtypeStruct((F, B, B), jnp.float32),
    )(ea2d, ep2d)


def kernel(X, emb_table, bias_table, w0):
    x2d = X.T.astype(jnp.int32)
    det3 = _sc_detile(emb_table.T).reshape(K, PSTRIDE, 128)
    tail = emb_table[999936:].T[:, None, :]
    det3 = lax.dynamic_update_slice(det3, tail, (0, 7812, 0))
    emb_pad = det3.reshape(K, PSTRIDE * 128)
    bias_lin = bias_table.reshape(V)
    w016 = jnp.broadcast_to(w0.astype(jnp.float32), (16,))
    ea2d, ep = _sc_gather_reduce(x2d, emb_pad, bias_lin, w016)
    out3 = _tc_broadcast(ea2d, ep.reshape(1, B))
    return out3.transpose(1, 0, 2)


# speedup vs baseline: 1.1036x; 1.0251x over previous
import functools

import jax
import jax.numpy as jnp
from jax import lax
from jax.experimental import pallas as pl
from jax.experimental.pallas import tpu as pltpu
from jax.experimental.pallas import tpu_sc as plsc

B = 1024
F = 26
K = 16
V = 1000000

NC = 2
NS = 16
NW = NC * NS
SAMP_PER_W = B // NW
IDX_PER_W = SAMP_PER_W * F
CHUNK = 104
NCHUNK = IDX_PER_W // CHUNK


TPW = 16
NWIN = 18
PSTRIDE = 7840


def _sc_detile_body(emb_hbm, out_hbm, b0, b1, b2, b3, b4, b5,
                    sem_r0, sem_r1, sem_r2, sem_w):
    wid = lax.axis_index("s") * NC + lax.axis_index("c")
    base = jnp.minimum(NWIN * wid, 488 - NWIN)
    rings = ((b0, b1), (b2, b3), (b4, b5))
    sem_r = (sem_r0, sem_r1, sem_r2)

    def _fire_reads(w, s):
        t0 = (base + w) * TPW
        for k1 in range(2):
            buf = rings[s][k1]
            for tl in range(TPW):
                col = pl.multiple_of((t0 + tl) * 128, 128)
                pltpu.async_copy(
                    emb_hbm.at[pl.ds(k1 * 8, 8), pl.ds(col, 128)],
                    buf.at[tl], sem_r[s])

    def _wait_reads(s):
        for _ in range(2 * TPW):
            pltpu.make_async_copy(emb_hbm.at[pl.ds(0, 8), pl.ds(0, 128)],
                                  rings[s][0].at[0], sem_r[s]).wait()

    def _drain_writes(n):
        for _ in range(16 * n):
            pltpu.make_async_copy(out_hbm.at[pl.ds(0, TPW)],
                                  b0.at[:, 0], sem_w).wait()

    _fire_reads(0, 0)
    _fire_reads(1, 1)

    def group(g, carry):
        for j in range(3):
            w = 3 * g + j
            s = j
            _wait_reads(s)
            for k1 in range(2):
                buf = rings[s][k1]
                for k2 in range(8):
                    row0 = (k1 * 8 + k2) * PSTRIDE + (base + w) * TPW
                    pltpu.async_copy(
                        buf.at[:, k2], out_hbm.at[pl.ds(row0, TPW)], sem_w)

            @pl.when(w >= 1)
            def _():
                _drain_writes(1)

            @pl.when(w + 2 < NWIN)
            def _():
                _fire_reads(w + 2, (j + 2) % 3)
        return carry

    lax.fori_loop(0, NWIN // 3, group, 0)
    _drain_writes(1)

    for k1 in range(2):
        buf = rings[0][k1]
        rds = []
        for tl in range(4):
            col = pl.multiple_of((7808 + tl) * 128, 128)
            rds.append(pltpu.async_copy(
                emb_hbm.at[pl.ds(k1 * 8, 8), pl.ds(col, 128)],
                buf.at[tl], sem_r0))
        for h in rds:
            h.wait()
        wrs = []
        for k2 in range(8):
            row0 = (k1 * 8 + k2) * PSTRIDE + 7808
            wrs.append(pltpu.async_copy(
                buf.at[:, k2], out_hbm.at[pl.ds(row0, TPW)], sem_w))
        for h in wrs:
            h.wait()


@jax.jit
def _sc_detile(emb_t):
    run = functools.partial(
        pl.kernel,
        mesh=plsc.VectorSubcoreMesh(core_axis_name="c", subcore_axis_name="s"),
        out_type=jax.ShapeDtypeStruct((K * PSTRIDE, 128), jnp.float32),
        scratch_types=[
            pltpu.VMEM((TPW, 8, 128), jnp.float32),
            pltpu.VMEM((TPW, 8, 128), jnp.float32),
            pltpu.VMEM((TPW, 8, 128), jnp.float32),
            pltpu.VMEM((TPW, 8, 128), jnp.float32),
            pltpu.VMEM((TPW, 8, 128), jnp.float32),
            pltpu.VMEM((TPW, 8, 128), jnp.float32),
            pltpu.SemaphoreType.DMA,
            pltpu.SemaphoreType.DMA,
            pltpu.SemaphoreType.DMA,
            pltpu.SemaphoreType.DMA,
        ],
        compiler_params=pltpu.CompilerParams(use_tc_tiling_on_sc=True),
    )(_sc_detile_body)
    return run(emb_t)


def _sc_body(x_hbm, emb_hbm, bias_hbm, w0_hbm, ea_out, ep_out,
             idx_v, val_v, bv_v, ea_v, ep_v, w0_v, sem):
    wid = lax.axis_index("s") * NC + lax.axis_index("c")
    sw = wid * SAMP_PER_W

    pltpu.sync_copy(w0_hbm, w0_v)
    idx_cps = [
        pltpu.async_copy(x_hbm.at[f, pl.ds(sw, SAMP_PER_W)],
                         idx_v.at[pl.ds(f * SAMP_PER_W, SAMP_PER_W)], sem)
        for f in range(F)
    ]
    for cp in idx_cps:
        cp.wait()

    copies = []
    for c in range(NCHUNK):
        sl = pl.ds(c * CHUNK, CHUNK)
        for k in range(K):
            copies.append(pltpu.async_copy(
                emb_hbm.at[k].at[idx_v.at[sl]],
                val_v.at[k, sl], sem))
        copies.append(pltpu.async_copy(
            bias_hbm.at[idx_v.at[sl]], bv_v.at[sl], sem))
    for cp in copies:
        cp.wait()

    w0v = w0_v[...]

    for h in range(SAMP_PER_W // 16):
        def eabody(f, carry):
            sl = pl.ds(f * SAMP_PER_W + h * 16, 16)
            ea_v[f, pl.ds(h * 16, 16)] = jnp.exp(-(w0v + bv_v[sl]))
            return carry
        lax.fori_loop(0, F, eabody, 0)

    for sb in range(SAMP_PER_W // 16):
        def fbody(f, accs):
            new = []
            for k in range(K):
                val = val_v[k, pl.ds(f * SAMP_PER_W + sb * 16, 16)]
                acc, asq = accs[2 * k], accs[2 * k + 1]
                new.append(acc + val)
                new.append(asq + val * val)
            return tuple(new)

        zero = jnp.zeros((16,), jnp.float32)
        accs = lax.fori_loop(0, F, fbody, (zero,) * (2 * K))
        u = zero
        for k in range(K):
            acc, asq = accs[2 * k], accs[2 * k + 1]
            u = u + (acc * acc - asq)
        ep_v[pl.ds(sb * 16, 16)] = jnp.exp(-0.5 * u)

    pltpu.sync_copy(ea_v, ea_out.at[:, pl.ds(sw, SAMP_PER_W)])
    pltpu.sync_copy(ep_v, ep_out.at[pl.ds(sw, SAMP_PER_W)])


@jax.jit
def _sc_gather_reduce(x2d, emb3, bias_lin, w016):
    run = functools.partial(
        pl.kernel,
        mesh=plsc.VectorSubcoreMesh(core_axis_name="c", subcore_axis_name="s"),
        out_type=[
            jax.ShapeDtypeStruct((F, B), jnp.float32),
            jax.ShapeDtypeStruct((B,), jnp.float32),
        ],
        scratch_types=[
            pltpu.VMEM((IDX_PER_W,), jnp.int32),
            pltpu.VMEM((K, IDX_PER_W), jnp.float32),
            pltpu.VMEM((IDX_PER_W,), jnp.float32),
            pltpu.VMEM((F, SAMP_PER_W), jnp.float32),
            pltpu.VMEM((SAMP_PER_W,), jnp.float32),
            pltpu.VMEM((16,), jnp.float32),
            pltpu.SemaphoreType.DMA,
        ],
        compiler_params=pltpu.CompilerParams(use_tc_tiling_on_sc=False),
    )(_sc_body)
    return run(x2d, emb3, bias_lin, w016)


BB = 128


def _tc_body(ea_ref, ep_ref, out_ref):
    ea = ea_ref[...]
    ep = ep_ref[...][0]
    x = ea[:, :, None] * ep[None, None, :]
    out_ref[...] = 5.5 / (1.0 + x)


@jax.jit
def _tc_broadcast(ea2d, ep2d):
    return pl.pallas_call(
        _tc_body,
        grid=(B // BB,),
        in_specs=[
            pl.BlockSpec((F, BB), lambda i: (0, i)),
            pl.BlockSpec((1, B), lambda i: (0, 0)),
        ],
        out_specs=pl.BlockSpec((F, BB, B), lambda i: (0, i, 0)),
        out_shape=jax.ShapeDtypeStruct((F, B, B), jnp.float32),
    )(ea2d, ep2d)


def kernel(X, emb_table, bias_table, w0):
    x2d = X.T.astype(jnp.int32)
    det3 = _sc_detile(emb_table.T).reshape(K, PSTRIDE, 128)
    tail = emb_table[999936:].T[:, None, :]
    det3 = lax.dynamic_update_slice(det3, tail, (0, 7812, 0))
    emb_pad = det3.reshape(K, PSTRIDE * 128)
    bias_lin = bias_table.reshape(V)
    w016 = jnp.broadcast_to(w0.astype(jnp.float32), (16,))
    ea2d, ep = _sc_gather_reduce(x2d, emb_pad, bias_lin, w016)
    out3 = _tc_broadcast(ea2d, ep.reshape(1, B))
    return out3.transpose(1, 0, 2)


# speedup vs baseline: 1.1601x; 1.0512x over previous
import functools

import jax
import jax.numpy as jnp
from jax import lax
from jax.experimental import pallas as pl
from jax.experimental.pallas import tpu as pltpu
from jax.experimental.pallas import tpu_sc as plsc

B = 1024
F = 26
K = 16
V = 1000000

NC = 2
NS = 16
NW = NC * NS
SAMP_PER_W = B // NW
IDX_PER_W = SAMP_PER_W * F
CHUNK = 104
NCHUNK = IDX_PER_W // CHUNK


TPW = 16
NWIN = 15
PSTRIDE = 7840


def _sc_detile_body(emb_hbm, out_hbm, b0, b1, b2, b3, b4, b5,
                    sem_r0, sem_r1, sem_r2, sem_w):
    wid = lax.axis_index("s") * NC + lax.axis_index("c")
    base = NWIN * wid
    rings = ((b0, b1), (b2, b3), (b4, b5))
    sem_r = (sem_r0, sem_r1, sem_r2)

    def _fire_reads(w, s):
        t0 = (base + w) * TPW
        for k1 in range(2):
            buf = rings[s][k1]
            for tl in range(TPW):
                col = pl.multiple_of((t0 + tl) * 128, 128)
                pltpu.async_copy(
                    emb_hbm.at[pl.ds(k1 * 8, 8), pl.ds(col, 128)],
                    buf.at[tl], sem_r[s])

    def _wait_reads(s):
        for _ in range(2 * TPW):
            pltpu.make_async_copy(emb_hbm.at[pl.ds(0, 8), pl.ds(0, 128)],
                                  rings[s][0].at[0], sem_r[s]).wait()

    def _drain_writes(n):
        for _ in range(16 * n):
            pltpu.make_async_copy(out_hbm.at[pl.ds(0, TPW)],
                                  b0.at[:, 0], sem_w).wait()

    _fire_reads(0, 0)
    _fire_reads(1, 1)

    def group(g, carry):
        for j in range(3):
            w = 3 * g + j
            s = j
            _wait_reads(s)
            for k1 in range(2):
                buf = rings[s][k1]
                for k2 in range(8):
                    row0 = (k1 * 8 + k2) * PSTRIDE + (base + w) * TPW
                    pltpu.async_copy(
                        buf.at[:, k2], out_hbm.at[pl.ds(row0, TPW)], sem_w)

            @pl.when(w >= 1)
            def _():
                _drain_writes(1)

            @pl.when(w + 2 < NWIN)
            def _():
                _fire_reads(w + 2, (j + 2) % 3)
        return carry

    lax.fori_loop(0, NWIN // 3, group, 0)
    _drain_writes(1)

    @pl.when(wid < 8)
    def _():
        t0 = (480 + wid) * TPW
        rds = []
        for k1 in range(2):
            buf = rings[0][k1]
            for tl in range(TPW):
                col = pl.multiple_of(t0 * 128 + tl * 128, 128)
                rds.append(pltpu.async_copy(
                    emb_hbm.at[pl.ds(k1 * 8, 8), pl.ds(col, 128)],
                    buf.at[tl], sem_r0))
        for h in rds:
            h.wait()
        wrs = []
        for k1 in range(2):
            buf = rings[0][k1]
            for k2 in range(8):
                row0 = (k1 * 8 + k2) * PSTRIDE + t0
                wrs.append(pltpu.async_copy(
                    buf.at[:, k2], out_hbm.at[pl.ds(row0, TPW)], sem_w))
        for h in wrs:
            h.wait()

    for k1 in range(2):
        buf = rings[0][k1]
        rds = []
        for tl in range(4):
            col = pl.multiple_of((7808 + tl) * 128, 128)
            rds.append(pltpu.async_copy(
                emb_hbm.at[pl.ds(k1 * 8, 8), pl.ds(col, 128)],
                buf.at[tl], sem_r0))
        for h in rds:
            h.wait()
        wrs = []
        for k2 in range(8):
            row0 = (k1 * 8 + k2) * PSTRIDE + 7808
            wrs.append(pltpu.async_copy(
                buf.at[:, k2], out_hbm.at[pl.ds(row0, TPW)], sem_w))
        for h in wrs:
            h.wait()


@jax.jit
def _sc_detile(emb_t):
    run = functools.partial(
        pl.kernel,
        mesh=plsc.VectorSubcoreMesh(core_axis_name="c", subcore_axis_name="s"),
        out_type=jax.ShapeDtypeStruct((K * PSTRIDE, 128), jnp.float32),
        scratch_types=[
            pltpu.VMEM((TPW, 8, 128), jnp.float32),
            pltpu.VMEM((TPW, 8, 128), jnp.float32),
            pltpu.VMEM((TPW, 8, 128), jnp.float32),
            pltpu.VMEM((TPW, 8, 128), jnp.float32),
            pltpu.VMEM((TPW, 8, 128), jnp.float32),
            pltpu.VMEM((TPW, 8, 128), jnp.float32),
            pltpu.SemaphoreType.DMA,
            pltpu.SemaphoreType.DMA,
            pltpu.SemaphoreType.DMA,
            pltpu.SemaphoreType.DMA,
        ],
        compiler_params=pltpu.CompilerParams(use_tc_tiling_on_sc=True),
    )(_sc_detile_body)
    return run(emb_t)


def _sc_body(x_hbm, emb_hbm, bias_hbm, w0_hbm, ea_out, ep_out,
             idx_v, val_v, bv_v, ea_v, ep_v, w0_v, sem):
    wid = lax.axis_index("s") * NC + lax.axis_index("c")
    sw = wid * SAMP_PER_W

    pltpu.sync_copy(w0_hbm, w0_v)
    idx_cps = [
        pltpu.async_copy(x_hbm.at[f, pl.ds(sw, SAMP_PER_W)],
                         idx_v.at[pl.ds(f * SAMP_PER_W, SAMP_PER_W)], sem)
        for f in range(F)
    ]
    for cp in idx_cps:
        cp.wait()

    copies = []
    for c in range(NCHUNK):
        sl = pl.ds(c * CHUNK, CHUNK)
        for k in range(K):
            copies.append(pltpu.async_copy(
                emb_hbm.at[k].at[idx_v.at[sl]],
                val_v.at[k, sl], sem))
        copies.append(pltpu.async_copy(
            bias_hbm.at[idx_v.at[sl]], bv_v.at[sl], sem))
    for cp in copies:
        cp.wait()

    w0v = w0_v[...]

    for h in range(SAMP_PER_W // 16):
        def eabody(f, carry):
            sl = pl.ds(f * SAMP_PER_W + h * 16, 16)
            ea_v[f, pl.ds(h * 16, 16)] = jnp.exp(-(w0v + bv_v[sl]))
            return carry
        lax.fori_loop(0, F, eabody, 0)

    for sb in range(SAMP_PER_W // 16):
        def fbody(f, accs):
            new = []
            for k in range(K):
                val = val_v[k, pl.ds(f * SAMP_PER_W + sb * 16, 16)]
                acc, asq = accs[2 * k], accs[2 * k + 1]
                new.append(acc + val)
                new.append(asq + val * val)
            return tuple(new)

        zero = jnp.zeros((16,), jnp.float32)
        accs = lax.fori_loop(0, F, fbody, (zero,) * (2 * K))
        u = zero
        for k in range(K):
            acc, asq = accs[2 * k], accs[2 * k + 1]
            u = u + (acc * acc - asq)
        ep_v[pl.ds(sb * 16, 16)] = jnp.exp(-0.5 * u)

    pltpu.sync_copy(ea_v, ea_out.at[:, pl.ds(sw, SAMP_PER_W)])
    pltpu.sync_copy(ep_v, ep_out.at[pl.ds(sw, SAMP_PER_W)])


@jax.jit
def _sc_gather_reduce(x2d, emb3, bias_lin, w016):
    run = functools.partial(
        pl.kernel,
        mesh=plsc.VectorSubcoreMesh(core_axis_name="c", subcore_axis_name="s"),
        out_type=[
            jax.ShapeDtypeStruct((F, B), jnp.float32),
            jax.ShapeDtypeStruct((B,), jnp.float32),
        ],
        scratch_types=[
            pltpu.VMEM((IDX_PER_W,), jnp.int32),
            pltpu.VMEM((K, IDX_PER_W), jnp.float32),
            pltpu.VMEM((IDX_PER_W,), jnp.float32),
            pltpu.VMEM((F, SAMP_PER_W), jnp.float32),
            pltpu.VMEM((SAMP_PER_W,), jnp.float32),
            pltpu.VMEM((16,), jnp.float32),
            pltpu.SemaphoreType.DMA,
        ],
        compiler_params=pltpu.CompilerParams(use_tc_tiling_on_sc=False),
    )(_sc_body)
    return run(x2d, emb3, bias_lin, w016)


BB = 128


def _tc_body(ea_ref, ep_ref, out_ref):
    ea = ea_ref[...]
    ep = ep_ref[...][0]
    x = ea[:, :, None] * ep[None, None, :]
    out_ref[...] = 5.5 / (1.0 + x)


@jax.jit
def _tc_broadcast(ea2d, ep2d):
    return pl.pallas_call(
        _tc_body,
        grid=(B // BB,),
        in_specs=[
            pl.BlockSpec((F, BB), lambda i: (0, i)),
            pl.BlockSpec((1, B), lambda i: (0, 0)),
        ],
        out_specs=pl.BlockSpec((F, BB, B), lambda i: (0, i, 0)),
        out_shape=jax.ShapeDtypeStruct((F, B, B), jnp.float32),
    )(ea2d, ep2d)


def kernel(X, emb_table, bias_table, w0):
    x2d = X.T.astype(jnp.int32)
    det3 = _sc_detile(emb_table.T).reshape(K, PSTRIDE, 128)
    tail = emb_table[999936:].T[:, None, :]
    det3 = lax.dynamic_update_slice(det3, tail, (0, 7812, 0))
    emb_pad = det3.reshape(K, PSTRIDE * 128)
    bias_lin = bias_table.reshape(V)
    w016 = jnp.broadcast_to(w0.astype(jnp.float32), (16,))
    ea2d, ep = _sc_gather_reduce(x2d, emb_pad, bias_lin, w016)
    out3 = _tc_broadcast(ea2d, ep.reshape(1, B))
    return out3.transpose(1, 0, 2)


# speedup vs baseline: 1.2187x; 1.0505x over previous
import functools

import jax
import jax.numpy as jnp
from jax import lax
from jax.experimental import pallas as pl
from jax.experimental.pallas import tpu as pltpu
from jax.experimental.pallas import tpu_sc as plsc

B = 1024
F = 26
K = 16
V = 1000000

NC = 2
NS = 16
NW = NC * NS
SAMP_PER_W = B // NW
IDX_PER_W = SAMP_PER_W * F
CHUNK = 104
NCHUNK = IDX_PER_W // CHUNK


TPW = 16
NWIN = 15
PSTRIDE = 7840


def _sc_detile_body(emb_hbm, out_hbm, b0, b1, b2, b3, b4, b5,
                    sem_r0, sem_r1, sem_r2, sem_w):
    wid = lax.axis_index("s") * NC + lax.axis_index("c")
    base = NWIN * wid
    rings = ((b0, b1), (b2, b3), (b4, b5))
    sem_r = (sem_r0, sem_r1, sem_r2)

    def _fire_reads(w, s):
        t0 = (base + w) * TPW
        for k1 in range(2):
            buf = rings[s][k1]
            for tl in range(TPW):
                col = pl.multiple_of((t0 + tl) * 128, 128)
                pltpu.async_copy(
                    emb_hbm.at[pl.ds(k1 * 8, 8), pl.ds(col, 128)],
                    buf.at[tl], sem_r[s])

    def _wait_reads(s):
        for _ in range(2 * TPW):
            pltpu.make_async_copy(emb_hbm.at[pl.ds(0, 8), pl.ds(0, 128)],
                                  rings[s][0].at[0], sem_r[s]).wait()

    def _drain_writes(n):
        for _ in range(16 * n):
            pltpu.make_async_copy(out_hbm.at[pl.ds(0, TPW)],
                                  b0.at[:, 0], sem_w).wait()

    _fire_reads(0, 0)
    _fire_reads(1, 1)

    def group(g, carry):
        for j in range(3):
            w = 3 * g + j
            s = j
            _wait_reads(s)
            for k1 in range(2):
                buf = rings[s][k1]
                for k2 in range(8):
                    row0 = (k1 * 8 + k2) * PSTRIDE + (base + w) * TPW
                    pltpu.async_copy(
                        buf.at[:, k2], out_hbm.at[pl.ds(row0, TPW)], sem_w)

            @pl.when(w >= 1)
            def _():
                _drain_writes(1)

            @pl.when(w + 2 < NWIN)
            def _():
                _fire_reads(w + 2, (j + 2) % 3)
        return carry

    lax.fori_loop(0, NWIN // 3, group, 0)
    _drain_writes(1)

    @pl.when(wid < 8)
    def _():
        t0 = (480 + wid) * TPW
        rds = []
        for k1 in range(2):
            buf = rings[0][k1]
            for tl in range(TPW):
                col = pl.multiple_of(t0 * 128 + tl * 128, 128)
                rds.append(pltpu.async_copy(
                    emb_hbm.at[pl.ds(k1 * 8, 8), pl.ds(col, 128)],
                    buf.at[tl], sem_r0))
        for h in rds:
            h.wait()
        wrs = []
        for k1 in range(2):
            buf = rings[0][k1]
            for k2 in range(8):
                row0 = (k1 * 8 + k2) * PSTRIDE + t0
                wrs.append(pltpu.async_copy(
                    buf.at[:, k2], out_hbm.at[pl.ds(row0, TPW)], sem_w))
        for h in wrs:
            h.wait()

    @pl.when(jnp.logical_and(wid >= 8, wid < 16))
    def _():
        for k1 in range(2):
            buf = rings[0][k1]
            rds = []
            for tl in range(4):
                col = pl.multiple_of((7808 + tl) * 128, 128)
                rds.append(pltpu.async_copy(
                    emb_hbm.at[pl.ds(k1 * 8, 8), pl.ds(col, 128)],
                    buf.at[tl], sem_r0))
            for h in rds:
                h.wait()
            wrs = []
            for k2 in range(8):
                row0 = (k1 * 8 + k2) * PSTRIDE + 7808
                wrs.append(pltpu.async_copy(
                    buf.at[:, k2], out_hbm.at[pl.ds(row0, TPW)], sem_w))
            for h in wrs:
                h.wait()


@jax.jit
def _sc_detile(emb_t):
    run = functools.partial(
        pl.kernel,
        mesh=plsc.VectorSubcoreMesh(core_axis_name="c", subcore_axis_name="s"),
        out_type=jax.ShapeDtypeStruct((K * PSTRIDE, 128), jnp.float32),
        scratch_types=[
            pltpu.VMEM((TPW, 8, 128), jnp.float32),
            pltpu.VMEM((TPW, 8, 128), jnp.float32),
            pltpu.VMEM((TPW, 8, 128), jnp.float32),
            pltpu.VMEM((TPW, 8, 128), jnp.float32),
            pltpu.VMEM((TPW, 8, 128), jnp.float32),
            pltpu.VMEM((TPW, 8, 128), jnp.float32),
            pltpu.SemaphoreType.DMA,
            pltpu.SemaphoreType.DMA,
            pltpu.SemaphoreType.DMA,
            pltpu.SemaphoreType.DMA,
        ],
        compiler_params=pltpu.CompilerParams(use_tc_tiling_on_sc=True),
    )(_sc_detile_body)
    return run(emb_t)


def _sc_body(x_hbm, emb_hbm, bias_hbm, w0_hbm, ea_out, ep_out,
             idx_v, val_v, bv_v, ea_v, ep_v, w0_v, sem):
    wid = lax.axis_index("s") * NC + lax.axis_index("c")
    sw = wid * SAMP_PER_W

    pltpu.sync_copy(w0_hbm, w0_v)
    idx_cps = [
        pltpu.async_copy(x_hbm.at[f, pl.ds(sw, SAMP_PER_W)],
                         idx_v.at[pl.ds(f * SAMP_PER_W, SAMP_PER_W)], sem)
        for f in range(F)
    ]
    for cp in idx_cps:
        cp.wait()

    copies = []
    for c in range(NCHUNK):
        sl = pl.ds(c * CHUNK, CHUNK)
        for k in range(K):
            copies.append(pltpu.async_copy(
                emb_hbm.at[k].at[idx_v.at[sl]],
                val_v.at[k, sl], sem))
        copies.append(pltpu.async_copy(
            bias_hbm.at[idx_v.at[sl]], bv_v.at[sl], sem))
    for cp in copies:
        cp.wait()

    w0v = w0_v[...]

    for h in range(SAMP_PER_W // 16):
        def eabody(f, carry):
            sl = pl.ds(f * SAMP_PER_W + h * 16, 16)
            ea_v[f, pl.ds(h * 16, 16)] = jnp.exp(-(w0v + bv_v[sl]))
            return carry
        lax.fori_loop(0, F, eabody, 0)

    for sb in range(SAMP_PER_W // 16):
        def fbody(f, accs):
            new = []
            for k in range(K):
                val = val_v[k, pl.ds(f * SAMP_PER_W + sb * 16, 16)]
                acc, asq = accs[2 * k], accs[2 * k + 1]
                new.append(acc + val)
                new.append(asq + val * val)
            return tuple(new)

        zero = jnp.zeros((16,), jnp.float32)
        accs = lax.fori_loop(0, F, fbody, (zero,) * (2 * K))
        u = zero
        for k in range(K):
            acc, asq = accs[2 * k], accs[2 * k + 1]
            u = u + (acc * acc - asq)
        ep_v[pl.ds(sb * 16, 16)] = jnp.exp(-0.5 * u)

    pltpu.sync_copy(ea_v, ea_out.at[:, pl.ds(sw, SAMP_PER_W)])
    pltpu.sync_copy(ep_v, ep_out.at[pl.ds(sw, SAMP_PER_W)])


@jax.jit
def _sc_gather_reduce(x2d, emb3, bias_lin, w016):
    run = functools.partial(
        pl.kernel,
        mesh=plsc.VectorSubcoreMesh(core_axis_name="c", subcore_axis_name="s"),
        out_type=[
            jax.ShapeDtypeStruct((F, B), jnp.float32),
            jax.ShapeDtypeStruct((B,), jnp.float32),
        ],
        scratch_types=[
            pltpu.VMEM((IDX_PER_W,), jnp.int32),
            pltpu.VMEM((K, IDX_PER_W), jnp.float32),
            pltpu.VMEM((IDX_PER_W,), jnp.float32),
            pltpu.VMEM((F, SAMP_PER_W), jnp.float32),
            pltpu.VMEM((SAMP_PER_W,), jnp.float32),
            pltpu.VMEM((16,), jnp.float32),
            pltpu.SemaphoreType.DMA,
        ],
        compiler_params=pltpu.CompilerParams(use_tc_tiling_on_sc=False),
    )(_sc_body)
    return run(x2d, emb3, bias_lin, w016)


BB = 128


def _tc_body(ea_ref, ep_ref, out_ref):
    ea = ea_ref[...]
    ep = ep_ref[...][0]
    x = ea[:, :, None] * ep[None, None, :]
    out_ref[...] = 5.5 / (1.0 + x)


@jax.jit
def _tc_broadcast(ea2d, ep2d):
    return pl.pallas_call(
        _tc_body,
        grid=(B // BB,),
        in_specs=[
            pl.BlockSpec((F, BB), lambda i: (0, i)),
            pl.BlockSpec((1, B), lambda i: (0, 0)),
        ],
        out_specs=pl.BlockSpec((F, BB, B), lambda i: (0, i, 0)),
        out_shape=jax.ShapeDtypeStruct((F, B, B), jnp.float32),
    )(ea2d, ep2d)


def kernel(X, emb_table, bias_table, w0):
    x2d = X.T.astype(jnp.int32)
    det3 = _sc_detile(emb_table.T).reshape(K, PSTRIDE, 128)
    tail = emb_table[999936:].T[:, None, :]
    det3 = lax.dynamic_update_slice(det3, tail, (0, 7812, 0))
    emb_pad = det3.reshape(K, PSTRIDE * 128)
    bias_lin = bias_table.reshape(V)
    w016 = jnp.broadcast_to(w0.astype(jnp.float32), (16,))
    ea2d, ep = _sc_gather_reduce(x2d, emb_pad, bias_lin, w016)
    out3 = _tc_broadcast(ea2d, ep.reshape(1, B))
    return out3.transpose(1, 0, 2)
